# baseline (speedup 1.0000x reference)
import functools

import jax
import jax.numpy as jnp
from jax import lax
from jax.experimental import pallas as pl
from jax.experimental.pallas import tpu as pltpu
from jax.experimental.pallas import tpu_sc as plsc

F = 26
V = 100001
D = 50
DP = 56
B = 16384
R = B * F
FV = F * V
TOTW = FV * D

_info = plsc.get_sparse_core_info()
NC, NS, L = _info.num_cores, _info.num_subcores, _info.num_lanes
NW = NC * NS

CR = 1024
CPW = 80
WR = CR * CPW
FVP = NW * WR
PARTIAL_RB = (FV // CR) * CR
PARTIAL_LEN = ((TOTW - PARTIAL_RB * D) // 8) * 8
TAIL_OFF = DP * (FV - 1) + 40

RW = R // NW
C = 512
NCHUNK = RW // C
KB = 128
KG = C // KB

_mesh = plsc.VectorSubcoreMesh(core_axis_name="c", subcore_axis_name="s")


@functools.partial(
    pl.kernel,
    mesh=_mesh,
    out_type=jax.ShapeDtypeStruct((FVP * DP,), jnp.float32),
    compiler_params=pltpu.CompilerParams(use_tc_tiling_on_sc=False),
    scratch_types=[
        pltpu.VMEM((CR * D + 16,), jnp.float32),
        pltpu.VMEM((CR * DP + 16,), jnp.float32),
        pltpu.VMEM((16,), jnp.float32),
    ],
)
def _repack_kernel(tab_hbm, tail_hbm, out_hbm, b50, b56, btail):
    wid = lax.axis_index("s") * NC + lax.axis_index("c")
    base = wid * WR

    def chunk(g, carry):
        rb = base + g * CR

        @pl.when(rb + CR <= FV)
        def _full():
            pltpu.sync_copy(tab_hbm.at[pl.ds(rb * D, CR * D)],
                            b50.at[pl.ds(0, CR * D)])

        @pl.when(rb == PARTIAL_RB)
        def _partial():
            pltpu.sync_copy(
                tab_hbm.at[pl.ds(PARTIAL_RB * D, PARTIAL_LEN)],
                b50.at[pl.ds(0, PARTIAL_LEN)],
            )

        @pl.when(rb < FV)
        def _emit():
            def repitch(j, c2):
                s = D * j
                t = DP * j
                b56[pl.ds(t, 16)] = b50[pl.ds(s, 16)]
                b56[pl.ds(t + 16, 16)] = b50[pl.ds(s + 16, 16)]
                b56[pl.ds(t + 32, 16)] = b50[pl.ds(s + 32, 16)]
                b56[pl.ds(t + 48, 16)] = b50[pl.ds(s + 48, 16)]
                return c2

            lax.fori_loop(0, CR, repitch, 0)
            pltpu.sync_copy(b56.at[pl.ds(0, CR * DP)],
                            out_hbm.at[pl.ds(rb * DP, CR * DP)])

        return carry

    lax.fori_loop(0, CPW, chunk, 0)

    @pl.when(wid == NW - 1)
    def _tail():
        pltpu.sync_copy(tail_hbm, btail)
        pltpu.sync_copy(btail, out_hbm.at[pl.ds(TAIL_OFF, 16)])


@functools.partial(
    pl.kernel,
    mesh=_mesh,
    out_type=jax.ShapeDtypeStruct((R, DP), jnp.float32),
    compiler_params=pltpu.CompilerParams(use_tc_tiling_on_sc=False),
    scratch_types=[
        pltpu.VMEM((KB,), jnp.int32),
        pltpu.VMEM((KB,), jnp.int32),
        pltpu.VMEM((KB,), jnp.int32),
        pltpu.VMEM((KB,), jnp.int32),
        pltpu.VMEM((C, DP), jnp.float32),
        pltpu.SemaphoreType.DMA,
    ],
)
def _gather_kernel(gidx_hbm, tab_hbm, out_hbm, g0, g1, g2, g3, rows_v, sem):
    gidx = [g0, g1, g2, g3]
    wid = lax.axis_index("s") * NC + lax.axis_index("c")
    base = wid * RW

    def chunk(g, carry):
        rowbase = base + g * C
        for k in range(KG):
            pltpu.sync_copy(gidx_hbm.at[pl.ds(rowbase + k * KB, KB)], gidx[k])
        copies = [
            pltpu.async_copy(
                tab_hbm.at[gidx[k]],
                rows_v.at[pl.ds(k * KB, KB)],
                sem,
            )
            for k in range(KG)
        ]
        for cp in copies:
            cp.wait()
        pltpu.sync_copy(rows_v, out_hbm.at[pl.ds(rowbase, C)])
        return carry

    lax.fori_loop(0, NCHUNK, chunk, 0)


def kernel(x, tables):
    offs = jnp.arange(F, dtype=jnp.int32) * V
    gidx = (x + offs[None, :]).reshape(R)
    tab_flat = tables.reshape(TOTW)
    tail = jnp.pad(tables[F - 1, V - 1, 40:50], (0, 6))
    tab56 = _repack_kernel(tab_flat, tail).reshape(FVP, DP)
    out = _gather_kernel(gidx, tab56)
    return out[:, :D].reshape(B, F * D)


# speedup vs baseline: 2.2137x; 2.2137x over previous
import functools

import jax
import jax.numpy as jnp
from jax import lax
from jax.experimental import pallas as pl
from jax.experimental.pallas import tpu as pltpu
from jax.experimental.pallas import tpu_sc as plsc

F = 26
V = 100001
D = 50
DP = 56
B = 16384
R = B * F

_info = plsc.get_sparse_core_info()
NC, NS, L = _info.num_cores, _info.num_subcores, _info.num_lanes
NW = NC * NS
RW = R // NW
C = 512
NCHUNK = RW // C
KB = 128
KG = C // KB

_mesh = plsc.VectorSubcoreMesh(core_axis_name="c", subcore_axis_name="s")


@functools.partial(
    pl.kernel,
    mesh=_mesh,
    out_type=jax.ShapeDtypeStruct((R, DP), jnp.float32),
    compiler_params=pltpu.CompilerParams(use_tc_tiling_on_sc=False),
    scratch_types=[
        pltpu.VMEM((KB,), jnp.int32),
        pltpu.VMEM((KB,), jnp.int32),
        pltpu.VMEM((KB,), jnp.int32),
        pltpu.VMEM((KB,), jnp.int32),
        pltpu.VMEM((C, DP), jnp.float32),
        pltpu.SemaphoreType.DMA,
    ],
)
def _gather_kernel(gidx_hbm, tab_hbm, out_hbm, g0, g1, g2, g3, rows_v, sem):
    gidx = [g0, g1, g2, g3]
    wid = lax.axis_index("s") * NC + lax.axis_index("c")
    base = wid * RW

    def chunk(g, carry):
        rowbase = base + g * C
        for k in range(KG):
            pltpu.sync_copy(gidx_hbm.at[pl.ds(rowbase + k * KB, KB)], gidx[k])
        copies = [
            pltpu.async_copy(
                tab_hbm.at[gidx[k]],
                rows_v.at[pl.ds(k * KB, KB)],
                sem,
            )
            for k in range(KG)
        ]
        for cp in copies:
            cp.wait()
        pltpu.sync_copy(rows_v, out_hbm.at[pl.ds(rowbase, C)])
        return carry

    lax.fori_loop(0, NCHUNK, chunk, 0)


def kernel(x, tables):
    offs = jnp.arange(F, dtype=jnp.int32) * V
    gidx = (x + offs[None, :]).reshape(R)
    tab_pad = lax.pad(tables.reshape(F * V, D), jnp.float32(0),
                      ((0, 0, 0), (0, DP - D, 0)))
    out = _gather_kernel(gidx, tab_pad)
    return out[:, :D].reshape(B, F * D)
